# Initial kernel scaffold; baseline (speedup 1.0000x reference)
import jax
import jax.numpy as jnp
from jax.experimental import pallas as pl


def kernel(data, neigh, depth):
    raise NotImplementedError("write your pallas kernel here")


# speedup vs baseline: 7.6707x; 7.6707x over previous
import functools

import jax
import jax.numpy as jnp
from jax import lax
from jax.experimental import pallas as pl
from jax.experimental.pallas import tpu as pltpu
from jax.experimental.pallas import tpu_sc as plsc

N_NODES = 50000
K = 27
C = 32
TOTAL = N_NODES * K
G = 80
NGRP = TOTAL // G
SB = 8
NW = 32
GRP_Q, GRP_R = divmod(NGRP, NW)
LANES = 16
PADROW = N_NODES


def _fix_indices(idx_v, base, count):
    for i in range(count // LANES):
        v = idx_v[pl.ds(base + i * LANES, LANES)]
        idx_v[pl.ds(base + i * LANES, LANES)] = jnp.where(v < 0, PADROW, v)


def _body(idx_hbm, data_hbm, out_hbm, idx_v, rows_v, sem):
    w = lax.axis_index("s") * 2 + lax.axis_index("c")
    ng = jnp.where(w < GRP_R, GRP_Q + 1, GRP_Q)
    base = w * GRP_Q + jnp.minimum(w, GRP_R)

    def super_chunk(a, carry):
        grp0 = base + a * SB
        pltpu.sync_copy(idx_hbm.at[pl.ds(grp0 * G, SB * G)], idx_v)
        _fix_indices(idx_v, 0, SB * G)
        copies = [
            pltpu.async_copy(
                data_hbm.at[idx_v.at[pl.ds(j * G, G)]],
                rows_v.at[pl.ds(j * G, G)],
                sem,
            )
            for j in range(SB)
        ]
        for cp in copies:
            cp.wait()
        pltpu.sync_copy(rows_v, out_hbm.at[pl.ds(grp0 * G, SB * G)])
        return carry

    def tail_group(t, carry):
        grp = base + (ng // SB) * SB + t
        pltpu.sync_copy(idx_hbm.at[pl.ds(grp * G, G)], idx_v.at[pl.ds(0, G)])
        _fix_indices(idx_v, 0, G)
        pltpu.async_copy(
            data_hbm.at[idx_v.at[pl.ds(0, G)]], rows_v.at[pl.ds(0, G)], sem
        ).wait()
        pltpu.sync_copy(rows_v.at[pl.ds(0, G)], out_hbm.at[pl.ds(grp * G, G)])
        return carry

    lax.fori_loop(0, ng // SB, super_chunk, 0)
    lax.fori_loop(0, ng % SB, tail_group, 0)


def kernel(data, neigh, depth):
    del depth
    data2 = jnp.concatenate([data, jnp.zeros((1, C), dtype=data.dtype)], axis=0)
    idx = neigh.astype(jnp.int32).reshape(TOTAL)

    mesh = plsc.VectorSubcoreMesh(core_axis_name="c", subcore_axis_name="s")
    run = functools.partial(
        pl.kernel,
        mesh=mesh,
        out_type=jax.ShapeDtypeStruct((TOTAL, C), jnp.float32),
        scratch_types=[
            pltpu.VMEM((SB * G,), jnp.int32),
            pltpu.VMEM((SB * G, C), jnp.float32),
            pltpu.SemaphoreType.DMA,
        ],
        compiler_params=pltpu.CompilerParams(use_tc_tiling_on_sc=False),
    )(_body)
    out = run(idx, data2)
    return out.reshape(N_NODES, K, C)
